# Initial kernel scaffold; baseline (speedup 1.0000x reference)
import functools

import jax
import jax.numpy as jnp
from jax import lax
from jax.experimental import pallas as pl
from jax.experimental.pallas import tpu as pltpu
from jax.experimental.pallas import tpu_sc as plsc

N = 10000
E = 320000
D = 128

NC = 2
NS = 16
E_PER_CORE = E // NC
E_PER_TILE = E_PER_CORE // NS
K = 80
NCHUNK = E_PER_TILE // K
ROWS_PER_TILE = N // NS
ZR = 25


def _sc_aggregate(cols, rows, vals, x):
    mesh = plsc.VectorSubcoreMesh(core_axis_name="c", subcore_axis_name="s")

    @functools.partial(
        pl.kernel,
        out_type=jax.ShapeDtypeStruct((NC, N, D), jnp.float32),
        mesh=mesh,
        scratch_types=[
            pltpu.VMEM((K,), jnp.int32),
            pltpu.VMEM((K,), jnp.int32),
            pltpu.VMEM((K,), jnp.float32),
            pltpu.VMEM((K, D), jnp.float32),
            pltpu.VMEM((ZR, D), jnp.float32),
            pltpu.VMEM_SHARED((N, D), jnp.float32),
            pltpu.SemaphoreType.DMA,
        ],
    )
    def body(cols_hbm, rows_hbm, vals_hbm, x_hbm, out_hbm,
             cols_v, rows_v, vals_v, gbuf, zbuf, agg_sh, sem):
        c = lax.axis_index("c")
        s = lax.axis_index("s")
        tile_base = c * E_PER_CORE + s * E_PER_TILE
        row0 = s * ROWS_PER_TILE

        zero16 = jnp.zeros((16,), jnp.float32)
        for r in range(ZR):
            for k8 in range(D // 16):
                zbuf[r, pl.ds(k8 * 16, 16)] = zero16

        def zero_slice(i, carry):
            pltpu.sync_copy(zbuf, agg_sh.at[pl.ds(row0 + i * ZR, ZR)])
            return carry
        lax.fori_loop(0, ROWS_PER_TILE // ZR, zero_slice, 0)
        plsc.subcore_barrier()

        def chunk(i, carry):
            base = tile_base + i * K
            pltpu.sync_copy(cols_hbm.at[pl.ds(base, K)], cols_v)
            pltpu.sync_copy(rows_hbm.at[pl.ds(base, K)], rows_v)
            pltpu.sync_copy(vals_hbm.at[pl.ds(base, K)], vals_v)
            pltpu.async_copy(x_hbm.at[cols_v], gbuf, sem).wait()

            def scale(j, inner):
                bv = plsc.load_gather(
                    vals_v, [jnp.full((16,), j, jnp.int32)])
                for k8 in range(D // 16):
                    seg = gbuf[j, pl.ds(k8 * 16, 16)]
                    gbuf[j, pl.ds(k8 * 16, 16)] = seg * bv
                return inner
            lax.fori_loop(0, K, scale, 0)

            pltpu.sync_copy(gbuf, agg_sh.at[rows_v], add=True)
            return carry
        lax.fori_loop(0, NCHUNK, chunk, 0)

        plsc.subcore_barrier()
        pltpu.sync_copy(agg_sh.at[pl.ds(row0, ROWS_PER_TILE)],
                        out_hbm.at[c, pl.ds(row0, ROWS_PER_TILE)])

    return body(cols, rows, vals, x)


BN = 2000


def _tc_linear_body(p0_ref, p1_ref, w_ref, b_ref, o_ref):
    agg = p0_ref[...] + p1_ref[...]
    w = w_ref[...]
    b = b_ref[...]
    dn = (((1,), (1,)), ((), ()))
    out1 = lax.dot_general(agg, w, dn, preferred_element_type=jnp.float32) + b
    o_ref[...] = lax.dot_general(
        out1, w, dn, preferred_element_type=jnp.float32) + b


def _tc_linear(p0, p1, W, b2d):
    return pl.pallas_call(
        _tc_linear_body,
        out_shape=jax.ShapeDtypeStruct((N, D), jnp.float32),
        grid=(N // BN,),
        in_specs=[
            pl.BlockSpec((BN, D), lambda i: (i, 0)),
            pl.BlockSpec((BN, D), lambda i: (i, 0)),
            pl.BlockSpec((D, D), lambda i: (0, 0)),
            pl.BlockSpec((1, D), lambda i: (0, 0)),
        ],
        out_specs=pl.BlockSpec((BN, D), lambda i: (i, 0)),
    )(p0, p1, W, b2d)


def kernel(edge_index, edge_values, x, W, b):
    rows = edge_index[0].astype(jnp.int32)
    cols = edge_index[1].astype(jnp.int32)
    partials = _sc_aggregate(cols, rows, edge_values, x)
    return _tc_linear(partials[0], partials[1], W, b.reshape(1, D))


# speedup vs baseline: 4.4558x; 4.4558x over previous
import functools

import jax
import jax.numpy as jnp
from jax import lax
from jax.experimental import pallas as pl
from jax.experimental.pallas import tpu as pltpu
from jax.experimental.pallas import tpu_sc as plsc

N = 10000
E = 320000
D = 128

NC = 2
NS = 16
E_PER_CORE = E // NC
E_PER_TILE = E_PER_CORE // NS
K = 80
NCHUNK = E_PER_TILE // K
RPT = 632
RPT_LAST = N - (NS - 1) * RPT
ZR = 8


def _lane_bcast(v, j):
    idx = jnp.full((16, 1), j, jnp.int32)
    return lax.gather(
        v, idx,
        lax.GatherDimensionNumbers(
            offset_dims=(), collapsed_slice_dims=(0,), start_index_map=(0,)),
        (1,), mode=lax.GatherScatterMode.PROMISE_IN_BOUNDS)


def _sc_aggregate(cols, rows, vals, x):
    mesh = plsc.VectorSubcoreMesh(core_axis_name="c", subcore_axis_name="s")

    @functools.partial(
        pl.kernel,
        out_type=jax.ShapeDtypeStruct((NC, N, D), jnp.float32),
        mesh=mesh,
        scratch_types=[
            pltpu.VMEM((K,), jnp.int32),
            pltpu.VMEM((K,), jnp.int32),
            pltpu.VMEM((K,), jnp.float32),
            pltpu.VMEM((K, D), jnp.float32),
            pltpu.VMEM((ZR, D), jnp.float32),
            pltpu.VMEM_SHARED((N, D), jnp.float32),
            pltpu.SemaphoreType.DMA,
        ],
    )
    def body(cols_hbm, rows_hbm, vals_hbm, x_hbm, out_hbm,
             cols_v, rows_v, vals_v, gbuf, zbuf, agg_sh, sem):
        c = lax.axis_index("c")
        s = lax.axis_index("s")
        tile_base = c * E_PER_CORE + s * E_PER_TILE
        row0 = pl.multiple_of(s * RPT, 8)
        nrow_blks = jnp.where(s < NS - 1, RPT // ZR, RPT_LAST // ZR)

        zero16 = jnp.zeros((16,), jnp.float32)
        for r in range(ZR):
            for k8 in range(D // 16):
                zbuf[r, pl.ds(k8 * 16, 16)] = zero16

        def zero_slice(i, carry):
            pltpu.sync_copy(
                zbuf, agg_sh.at[pl.ds(pl.multiple_of(row0 + i * ZR, 8), ZR)])
            return carry
        lax.fori_loop(0, nrow_blks, zero_slice, 0)
        plsc.subcore_barrier()

        def chunk(i, carry):
            base = tile_base + i * K
            pltpu.sync_copy(cols_hbm.at[pl.ds(base, K)], cols_v)
            pltpu.sync_copy(rows_hbm.at[pl.ds(base, K)], rows_v)
            pltpu.sync_copy(vals_hbm.at[pl.ds(base, K)], vals_v)
            pltpu.async_copy(x_hbm.at[cols_v], gbuf, sem).wait()

            def scale_grp(g, inner):
                vv = vals_v[pl.ds(pl.multiple_of(g * 16, 16), 16)]
                for j in range(16):
                    bv = _lane_bcast(vv, j)
                    e = g * 16 + j
                    for k8 in range(D // 16):
                        seg = gbuf[e, pl.ds(k8 * 16, 16)]
                        gbuf[e, pl.ds(k8 * 16, 16)] = seg * bv
                return inner
            lax.fori_loop(0, K // 16, scale_grp, 0)

            pltpu.sync_copy(gbuf, agg_sh.at[rows_v], add=True)
            return carry
        lax.fori_loop(0, NCHUNK, chunk, 0)

        plsc.subcore_barrier()

        @pl.when(s < NS - 1)
        def _copy_main():
            pltpu.sync_copy(agg_sh.at[pl.ds(row0, RPT)],
                            out_hbm.at[c, pl.ds(row0, RPT)])

        @pl.when(s == NS - 1)
        def _copy_last():
            pltpu.sync_copy(agg_sh.at[pl.ds(row0, RPT_LAST)],
                            out_hbm.at[c, pl.ds(row0, RPT_LAST)])

    return body(cols, rows, vals, x)


BN = 2000


def _tc_linear_body(p0_ref, p1_ref, w_ref, b_ref, o_ref):
    agg = p0_ref[...] + p1_ref[...]
    w = w_ref[...]
    b = b_ref[...]
    dn = (((1,), (1,)), ((), ()))
    out1 = lax.dot_general(agg, w, dn, preferred_element_type=jnp.float32) + b
    o_ref[...] = lax.dot_general(
        out1, w, dn, preferred_element_type=jnp.float32) + b


def _tc_linear(p0, p1, W, b2d):
    return pl.pallas_call(
        _tc_linear_body,
        out_shape=jax.ShapeDtypeStruct((N, D), jnp.float32),
        grid=(N // BN,),
        in_specs=[
            pl.BlockSpec((BN, D), lambda i: (i, 0)),
            pl.BlockSpec((BN, D), lambda i: (i, 0)),
            pl.BlockSpec((D, D), lambda i: (0, 0)),
            pl.BlockSpec((1, D), lambda i: (0, 0)),
        ],
        out_specs=pl.BlockSpec((BN, D), lambda i: (i, 0)),
    )(p0, p1, W, b2d)


def kernel(edge_index, edge_values, x, W, b):
    rows = edge_index[0].astype(jnp.int32)
    cols = edge_index[1].astype(jnp.int32)
    partials = _sc_aggregate(cols, rows, edge_values, x)
    return _tc_linear(partials[0], partials[1], W, b.reshape(1, D))


# speedup vs baseline: 4.4858x; 1.0068x over previous
import functools

import jax
import jax.numpy as jnp
from jax import lax
from jax.experimental import pallas as pl
from jax.experimental.pallas import tpu as pltpu
from jax.experimental.pallas import tpu_sc as plsc

N = 10000
E = 320000
D = 128

NC = 2
NS = 16
NW = NC * NS
K = 128
CPT = 80
CPW = 16
NWIN = CPT // CPW
EW = CPW * K
E_PER_TILE = K * CPT
E_PAD = NW * E_PER_TILE
RPT = 632
RPT_LAST = N - (NS - 1) * RPT
ZR = 8


def _lane_bcast(v, j):
    idx = jnp.full((16, 1), j, jnp.int32)
    return lax.gather(
        v, idx,
        lax.GatherDimensionNumbers(
            offset_dims=(), collapsed_slice_dims=(0,), start_index_map=(0,)),
        (1,), mode=lax.GatherScatterMode.PROMISE_IN_BOUNDS)


def _sc_aggregate(cols, rows2d, vals, x):
    mesh = plsc.VectorSubcoreMesh(core_axis_name="c", subcore_axis_name="s")

    @functools.partial(
        pl.kernel,
        out_type=jax.ShapeDtypeStruct((NC, N, D), jnp.float32),
        mesh=mesh,
        scratch_types=[
            pltpu.VMEM((EW,), jnp.int32),
            pltpu.VMEM((EW,), jnp.int32),
            pltpu.VMEM((EW,), jnp.float32),
            pltpu.VMEM((EW,), jnp.float32),
            pltpu.VMEM((CPW, K), jnp.int32),
            pltpu.VMEM((CPW, K), jnp.int32),
            pltpu.VMEM((K, D), jnp.float32),
            pltpu.VMEM((K, D), jnp.float32),
            pltpu.VMEM_SHARED((N, D), jnp.float32),
            pltpu.SemaphoreType.DMA,
            pltpu.SemaphoreType.DMA,
            pltpu.SemaphoreType.DMA,
        ],
    )
    def body(cols_hbm, rows_hbm, vals_hbm, x_hbm, out_hbm,
             colsA, colsB, valsA, valsB, rowsA, rowsB, gbuf0, gbuf1,
             agg_sh, sem0, sem1, semw):
        c = lax.axis_index("c")
        s = lax.axis_index("s")
        wid = c * NS + s
        row0 = pl.multiple_of(s * RPT, 8)
        nrow_blks = jnp.where(s < NS - 1, RPT // ZR, RPT_LAST // ZR)

        cbufs = (colsA, colsB)
        vbufs = (valsA, valsB)
        rbufs = (rowsA, rowsB)

        def win_load(w, sem):
            ebase = wid * E_PER_TILE + w * EW
            cbase = wid * CPT + w * CPW
            cw, vw, rw = cbufs[w % 2], vbufs[w % 2], rbufs[w % 2]
            return (
                pltpu.async_copy(cols_hbm.at[pl.ds(ebase, EW)], cw, sem),
                pltpu.async_copy(vals_hbm.at[pl.ds(ebase, EW)], vw, sem),
                pltpu.async_copy(rows_hbm.at[pl.ds(cbase, CPW)], rw, sem),
            )

        descs = win_load(0, semw)

        zero16 = jnp.zeros((16,), jnp.float32)
        for r in range(ZR):
            for k8 in range(D // 16):
                gbuf0[r, pl.ds(k8 * 16, 16)] = zero16

        def zero_slice(i, carry):
            pltpu.sync_copy(
                gbuf0.at[pl.ds(0, ZR)],
                agg_sh.at[pl.ds(pl.multiple_of(row0 + i * ZR, 8), ZR)])
            return carry
        lax.fori_loop(0, nrow_blks, zero_slice, 0)
        plsc.subcore_barrier()

        def gather(cw, i, buf, sem):
            return pltpu.async_copy(
                x_hbm.at[cw.at[pl.ds(i * K, K)]], buf, sem)

        def scale(vw, chunk, buf):
            def grp(g, inner):
                vv = vw[pl.ds(chunk * K + g * 16, 16)]
                for j in range(16):
                    bv = _lane_bcast(vv, j)
                    e = g * 16 + j
                    for k8 in range(D // 16):
                        seg = buf[e, pl.ds(k8 * 16, 16)]
                        buf[e, pl.ds(k8 * 16, 16)] = seg * bv
                return inner
            lax.fori_loop(0, K // 16, grp, 0)

        for d in descs:
            d.wait()
        gather(cbufs[0], 0, gbuf0, sem0)

        for w in range(NWIN):
            cw, vw, rw = cbufs[w % 2], vbufs[w % 2], rbufs[w % 2]
            if w + 1 < NWIN:
                descs = win_load(w + 1, semw)

            def pair(t, carry, cw=cw, vw=vw, rw=rw):
                i0 = t * 2
                pltpu.make_async_copy(
                    x_hbm.at[cw.at[pl.ds(i0 * K, K)]], gbuf0, sem0).wait()
                gather(cw, i0 + 1, gbuf1, sem1)
                scale(vw, i0, gbuf0)
                pltpu.sync_copy(gbuf0, agg_sh.at[rw.at[i0]], add=True)

                pltpu.make_async_copy(
                    x_hbm.at[cw.at[pl.ds((i0 + 1) * K, K)]], gbuf1,
                    sem1).wait()

                @pl.when(t < CPW // 2 - 1)
                def _prefetch():
                    gather(cw, i0 + 2, gbuf0, sem0)
                scale(vw, i0 + 1, gbuf1)
                pltpu.sync_copy(gbuf1, agg_sh.at[rw.at[i0 + 1]], add=True)
                return carry
            lax.fori_loop(0, CPW // 2, pair, 0)

            if w + 1 < NWIN:
                for d in descs:
                    d.wait()
                gather(cbufs[(w + 1) % 2], 0, gbuf0, sem0)

        plsc.subcore_barrier()

        @pl.when(s < NS - 1)
        def _copy_main():
            pltpu.sync_copy(agg_sh.at[pl.ds(row0, RPT)],
                            out_hbm.at[c, pl.ds(row0, RPT)])

        @pl.when(s == NS - 1)
        def _copy_last():
            pltpu.sync_copy(agg_sh.at[pl.ds(row0, RPT_LAST)],
                            out_hbm.at[c, pl.ds(row0, RPT_LAST)])

    return body(cols, rows2d, vals, x)


BN = 2000


def _tc_linear_body(p0_ref, p1_ref, w_ref, b_ref, o_ref):
    agg = p0_ref[...] + p1_ref[...]
    w = w_ref[...]
    b = b_ref[...]
    dn = (((1,), (1,)), ((), ()))
    out1 = lax.dot_general(agg, w, dn, preferred_element_type=jnp.float32) + b
    o_ref[...] = lax.dot_general(
        out1, w, dn, preferred_element_type=jnp.float32) + b


def _tc_linear(p0, p1, W, b2d):
    return pl.pallas_call(
        _tc_linear_body,
        out_shape=jax.ShapeDtypeStruct((N, D), jnp.float32),
        grid=(N // BN,),
        in_specs=[
            pl.BlockSpec((BN, D), lambda i: (i, 0)),
            pl.BlockSpec((BN, D), lambda i: (i, 0)),
            pl.BlockSpec((D, D), lambda i: (0, 0)),
            pl.BlockSpec((1, D), lambda i: (0, 0)),
        ],
        out_specs=pl.BlockSpec((BN, D), lambda i: (i, 0)),
    )(p0, p1, W, b2d)


def kernel(edge_index, edge_values, x, W, b):
    rows = edge_index[0].astype(jnp.int32)
    cols = edge_index[1].astype(jnp.int32)
    pad = E_PAD - E
    rows2d = jnp.pad(rows, (0, pad)).reshape(E_PAD // K, K)
    cols_p = jnp.pad(cols, (0, pad))
    vals_p = jnp.pad(edge_values, (0, pad))
    partials = _sc_aggregate(cols_p, rows2d, vals_p, x)
    return _tc_linear(partials[0], partials[1], W, b.reshape(1, D))


# speedup vs baseline: 5.6027x; 1.2490x over previous
import functools

import jax
import jax.numpy as jnp
from jax import lax
from jax.experimental import pallas as pl
from jax.experimental.pallas import tpu as pltpu
from jax.experimental.pallas import tpu_sc as plsc

N = 10000
E = 320000
D = 128

NC = 2
NS = 16
NW = NC * NS
K = 32
CPT = 320
CPW = 4
NWIN = CPT // CPW
EW = CPW * K
E_PER_TILE = K * CPT
E_PAD = NW * E_PER_TILE

NPASS = 2
HALF = N // NPASS
AGGR = HALF + 8

RPT = 632
RPT_LAST = N - (NS - 1) * RPT
ZPT = 312
ZPT_LAST = AGGR - (NS - 1) * ZPT
CPT_LAST = HALF - (NS - 1) * ZPT
ZR = 8


def _lane_bcast(v, j):
    idx = jnp.full((16, 1), j, jnp.int32)
    return lax.gather(
        v, idx,
        lax.GatherDimensionNumbers(
            offset_dims=(), collapsed_slice_dims=(0,), start_index_map=(0,)),
        (1,), mode=lax.GatherScatterMode.PROMISE_IN_BOUNDS)


def _sc_aggregate(cols, rows2d, vals, x):
    mesh = plsc.VectorSubcoreMesh(core_axis_name="c", subcore_axis_name="s")

    @functools.partial(
        pl.kernel,
        out_type=jax.ShapeDtypeStruct((NC, N, D), jnp.float32),
        mesh=mesh,
        scratch_types=[
            pltpu.VMEM((EW,), jnp.int32),
            pltpu.VMEM((EW,), jnp.int32),
            pltpu.VMEM((EW,), jnp.float32),
            pltpu.VMEM((EW,), jnp.float32),
            pltpu.VMEM((CPW, K), jnp.int32),
            pltpu.VMEM((CPW, K), jnp.int32),
            pltpu.VMEM((K, D), jnp.float32),
            pltpu.VMEM((K, D), jnp.float32),
            pltpu.VMEM((K,), jnp.int32),
            pltpu.VMEM_SHARED((N, D), jnp.float32),
            pltpu.VMEM_SHARED((AGGR, D), jnp.float32),
            pltpu.SemaphoreType.DMA,
            pltpu.SemaphoreType.DMA,
            pltpu.SemaphoreType.DMA,
        ],
    )
    def body(cols_hbm, rows_hbm, vals_hbm, x_hbm, out_hbm,
             colsA, colsB, valsA, valsB, rowsA, rowsB, gbuf0, gbuf1,
             ridx, x_sp, agg_sh, sem0, sem1, semw):
        c = lax.axis_index("c")
        s = lax.axis_index("s")
        wid = c * NS + s
        xrow0 = pl.multiple_of(s * RPT, 8)
        arow0 = pl.multiple_of(s * ZPT, 8)
        nzero_blks = jnp.where(s < NS - 1, ZPT // ZR, ZPT_LAST // ZR)

        cbufs = (colsA, colsB)
        vbufs = (valsA, valsB)
        rbufs = (rowsA, rowsB)
        gbufs = (gbuf0, gbuf1)
        gsems = (sem0, sem1)

        @pl.when(s < NS - 1)
        def _stage_main():
            pltpu.sync_copy(x_hbm.at[pl.ds(xrow0, RPT)],
                            x_sp.at[pl.ds(xrow0, RPT)])

        @pl.when(s == NS - 1)
        def _stage_last():
            pltpu.sync_copy(x_hbm.at[pl.ds(xrow0, RPT_LAST)],
                            x_sp.at[pl.ds(xrow0, RPT_LAST)])

        def win_load(w, parity, sem):
            ebase = wid * E_PER_TILE + w * EW
            cbase = wid * CPT + w * CPW
            cw, vw, rw = cbufs[parity], vbufs[parity], rbufs[parity]
            return (
                pltpu.async_copy(cols_hbm.at[pl.ds(ebase, EW)], cw, sem),
                pltpu.async_copy(vals_hbm.at[pl.ds(ebase, EW)], vw, sem),
                pltpu.async_copy(rows_hbm.at[pl.ds(cbase, CPW)], rw, sem),
            )

        def gather(cw, i, b):
            return pltpu.async_copy(
                x_sp.at[cw.at[pl.ds(i * K, K)]], gbufs[b], gsems[b])

        def gwait(cw, i, b):
            pltpu.make_async_copy(
                x_sp.at[cw.at[pl.ds(i * K, K)]], gbufs[b], gsems[b]).wait()

        def scale(vw, chunk, buf):
            def grp(g, inner):
                vv = vw[pl.ds(chunk * K + g * 16, 16)]
                for j in range(16):
                    bv = _lane_bcast(vv, j)
                    e = g * 16 + j
                    for k8 in range(D // 16):
                        seg = buf[e, pl.ds(k8 * 16, 16)]
                        buf[e, pl.ds(k8 * 16, 16)] = seg * bv
                return inner
            lax.fori_loop(0, K // 16, grp, 0)

        lanes8 = lax.iota(jnp.int32, 16) & 7

        for p in range(NPASS):
            lo = p * HALF

            zero16 = jnp.zeros((16,), jnp.float32)
            for r in range(ZR):
                for k8 in range(D // 16):
                    gbuf0[r, pl.ds(k8 * 16, 16)] = zero16

            def zero_slice(i, carry):
                pltpu.sync_copy(
                    gbuf0.at[pl.ds(0, ZR)],
                    agg_sh.at[pl.ds(pl.multiple_of(arow0 + i * ZR, 8), ZR)])
                return carry
            lax.fori_loop(0, nzero_blks, zero_slice, 0)
            plsc.subcore_barrier()

            def run_window(parity, w, last_chunk_prefetch, lo):
                cw, vw, rw = cbufs[parity], vbufs[parity], rbufs[parity]

                def pair(t, carry):
                    for b in range(2):
                        i = t * 2 + b
                        gwait(cw, i, b)

                        @pl.when(jnp.logical_or(t < CPW // 2 - 1, b == 0))
                        def _pref():
                            gather(cw, i + 1, 1 - b)
                        scale(vw, i, gbufs[b])
                        for h in range(K // 16):
                            rr = rw[i, pl.ds(h * 16, 16)]
                            t_ = rr - lo
                            ok = jnp.logical_and(t_ >= 0, t_ < HALF)
                            ridx[pl.ds(h * 16, 16)] = jnp.where(
                                ok, t_, HALF + lanes8)
                        pltpu.sync_copy(
                            gbufs[b], agg_sh.at[ridx], add=True)
                    return carry
                lax.fori_loop(0, CPW // 2, pair, 0)

            descs = win_load(0, 0, semw)
            for d in descs:
                d.wait()
            gather(cbufs[0], 0, 0)

            def winpair(u, carry, lo=lo):
                wA = u * 2
                dB = win_load(wA + 1, 1, semw)
                run_window(0, wA, None, lo)
                for d in dB:
                    d.wait()
                gather(cbufs[1], 0, 0)

                @pl.when(u < NWIN // 2 - 1)
                def _load_next():
                    win_load(wA + 2, 0, semw)
                run_window(1, wA + 1, None, lo)

                @pl.when(u < NWIN // 2 - 1)
                def _wait_next():
                    ebase = wid * E_PER_TILE + (wA + 2) * EW
                    cbase = wid * CPT + (wA + 2) * CPW
                    pltpu.make_async_copy(
                        cols_hbm.at[pl.ds(ebase, EW)], cbufs[0], semw).wait()
                    pltpu.make_async_copy(
                        vals_hbm.at[pl.ds(ebase, EW)], vbufs[0], semw).wait()
                    pltpu.make_async_copy(
                        rows_hbm.at[pl.ds(cbase, CPW)], rbufs[0], semw).wait()
                    gather(cbufs[0], 0, 0)
                return carry
            lax.fori_loop(0, NWIN // 2, winpair, 0)

            plsc.subcore_barrier()

            @pl.when(s < NS - 1)
            def _copy_main():
                pltpu.sync_copy(
                    agg_sh.at[pl.ds(arow0, ZPT)],
                    out_hbm.at[c, pl.ds(pl.multiple_of(lo + arow0, 8), ZPT)])

            @pl.when(s == NS - 1)
            def _copy_last():
                pltpu.sync_copy(
                    agg_sh.at[pl.ds(arow0, CPT_LAST)],
                    out_hbm.at[c, pl.ds(pl.multiple_of(lo + arow0, 8),
                                        CPT_LAST)])

    return body(cols, rows2d, vals, x)


BN = 2000


def _tc_linear_body(p0_ref, p1_ref, w_ref, b_ref, o_ref):
    agg = p0_ref[...] + p1_ref[...]
    w = w_ref[...]
    b = b_ref[...]
    dn = (((1,), (1,)), ((), ()))
    out1 = lax.dot_general(agg, w, dn, preferred_element_type=jnp.float32) + b
    o_ref[...] = lax.dot_general(
        out1, w, dn, preferred_element_type=jnp.float32) + b


def _tc_linear(p0, p1, W, b2d):
    return pl.pallas_call(
        _tc_linear_body,
        out_shape=jax.ShapeDtypeStruct((N, D), jnp.float32),
        grid=(N // BN,),
        in_specs=[
            pl.BlockSpec((BN, D), lambda i: (i, 0)),
            pl.BlockSpec((BN, D), lambda i: (i, 0)),
            pl.BlockSpec((D, D), lambda i: (0, 0)),
            pl.BlockSpec((1, D), lambda i: (0, 0)),
        ],
        out_specs=pl.BlockSpec((BN, D), lambda i: (i, 0)),
    )(p0, p1, W, b2d)


def kernel(edge_index, edge_values, x, W, b):
    rows = edge_index[0].astype(jnp.int32)
    cols = edge_index[1].astype(jnp.int32)
    pad = E_PAD - E
    rows2d = jnp.pad(rows, (0, pad)).reshape(E_PAD // K, K)
    cols_p = jnp.pad(cols, (0, pad))
    vals_p = jnp.pad(edge_values, (0, pad))
    partials = _sc_aggregate(cols_p, rows2d, vals_p, x)
    return _tc_linear(partials[0], partials[1], W, b.reshape(1, D))
